# baseline (speedup 1.0000x reference)
import functools

import jax
import jax.numpy as jnp
from jax import lax
from jax.experimental import pallas as pl
from jax.experimental.pallas import tpu as pltpu
from jax.experimental.pallas import tpu_sc as plsc

_B, _S = 16384, 50
_D = 32
_N_IDX = _B * _S
_CHUNK = 128
_N_ROWS = _N_IDX // _CHUNK

_info = plsc.get_sparse_core_info()
_NC, _NS = _info.num_cores, _info.num_subcores
_NW = _NC * _NS

_ROWS_PER_W = _N_ROWS // _NW
_K = 10
_STEP_ROWS = _K * _CHUNK
_N_STEPS = _ROWS_PER_W // _K
_N_OUTER = _N_STEPS // 2


def _emb_body(table, idx, out, idx_v, rows_v, sem0, sem1):
    wid = lax.axis_index("s") * _NC + lax.axis_index("c")
    base_row = wid * _ROWS_PER_W
    sems = (sem0, sem1)

    pltpu.sync_copy(idx.at[pl.ds(base_row, _ROWS_PER_W)], idx_v)

    def fire(s, b):
        for j in range(_K):
            pltpu.async_copy(
                table.at[idx_v.at[s * _K + j]],
                rows_v.at[b, pl.ds(j * _CHUNK, _CHUNK)],
                sems[b],
            )

    def drain_store(s, b):
        pltpu.make_async_copy(
            table.at[pl.ds(0, _STEP_ROWS)], rows_v.at[b], sems[b]
        ).wait()
        pltpu.sync_copy(
            rows_v.at[b],
            out.at[pl.ds((base_row + s * _K) * _CHUNK, _STEP_ROWS)],
        )

    fire(0, 0)

    def outer(t, carry):
        s0 = 2 * t
        fire(s0 + 1, 1)
        drain_store(s0, 0)

        @pl.when(t < _N_OUTER - 1)
        def _():
            fire(s0 + 2, 0)

        drain_store(s0 + 1, 1)
        return carry

    lax.fori_loop(0, _N_OUTER, outer, 0)


@functools.partial(
    pl.kernel,
    mesh=plsc.VectorSubcoreMesh(core_axis_name="c", subcore_axis_name="s"),
    out_type=jax.ShapeDtypeStruct((_N_IDX, _D), jnp.float32),
    scratch_types=[
        pltpu.VMEM((_ROWS_PER_W, _CHUNK), jnp.int32),
        pltpu.VMEM((2, _STEP_ROWS, _D), jnp.float32),
        pltpu.SemaphoreType.DMA,
        pltpu.SemaphoreType.DMA,
    ],
    compiler_params=pltpu.CompilerParams(use_tc_tiling_on_sc=False),
)
def _emb_kernel(table, idx, out, idx_v, rows_v, sem0, sem1):
    _emb_body(table, idx, out, idx_v, rows_v, sem0, sem1)


def kernel(token_ids, weight):
    idx = token_ids.astype(jnp.int32).reshape(_N_ROWS, _CHUNK)
    out = _emb_kernel(weight, idx)
    return out.reshape(_B, _S, _D)
